# baseline (speedup 1.0000x reference)
import functools

import jax
import jax.numpy as jnp
from jax import lax
from jax.experimental import pallas as pl
from jax.experimental.pallas import tpu as pltpu
from jax.experimental.pallas import tpu_sc as plsc

_NC, _NS, _LANES = 2, 16, 16
_NW = _NC * _NS

_NG = 3
_CH = 8
_UNROLL = 4

_SC_FRAC = 0.375
_TC_BS = 640


def _sc_add(nbatch, nseq, row0, nrows, d):
    seq_per_w = nrows // _NW
    nchunk = seq_per_w // _CH
    assert nchunk * _CH * _NW == nrows
    assert d & (d - 1) == 0
    dshift = d.bit_length() - 1

    mesh = plsc.VectorSubcoreMesh(core_axis_name="c", subcore_axis_name="s")

    @functools.partial(
        pl.kernel,
        mesh=mesh,
        out_type=jax.ShapeDtypeStruct((nbatch * nrows, d), jnp.float32),
        scratch_types=[
            pltpu.VMEM((_NG, nbatch, _CH, d), jnp.float32),
            pltpu.VMEM((_NG, _CH, d), jnp.float32),
        ] + [pltpu.SemaphoreType.DMA] * (3 * _NG),
    )
    def body(x_hbm, p_hbm, o_hbm, xbuf, pbuf, *sems):
        lsems = sems[:_NG]
        psems = sems[_NG:2 * _NG]
        ssems = sems[2 * _NG:]
        c = lax.axis_index("c")
        s = lax.axis_index("s")
        wid = s * _NC + c
        base = wid * seq_per_w

        def issue_group(k):
            g = k % _NG
            hs = [pltpu.async_copy(
                p_hbm.at[pl.ds(row0 + base + k * _CH, _CH)],
                pbuf.at[g], psems[g])]
            for b in range(nbatch):
                roff = b * nseq + row0 + base + k * _CH
                hs.append(pltpu.async_copy(x_hbm.at[pl.ds(roff, _CH)],
                                           xbuf.at[g, b], lsems[g]))
            return hs

        loads = {k: issue_group(k) for k in range(min(_NG - 1, nchunk))}
        stores = {}
        for k in range(nchunk):
            g = k % _NG
            for h in loads.pop(k):
                h.wait()

            @plsc.parallel_loop(0, _CH * d, step=_LANES, unroll=_UNROLL)
            def cbody(o, g=g):
                r = o >> dshift
                sl = pl.ds(pl.multiple_of(o & (d - 1), _LANES), _LANES)
                p = pbuf[g, r, sl]
                for b in range(nbatch):
                    xbuf[g, b, r, sl] = xbuf[g, b, r, sl] + p

            shs = []
            for b in range(nbatch):
                roff = b * nrows + base + k * _CH
                shs.append(pltpu.async_copy(xbuf.at[g, b],
                                            o_hbm.at[pl.ds(roff, _CH)],
                                            ssems[g]))
            stores[k] = shs

            kn = k + _NG - 1
            if kn < nchunk:
                if k - 1 >= 0:
                    for h in stores.pop(k - 1):
                        h.wait()
                loads[kn] = issue_group(kn)
        for k in sorted(stores):
            for h in stores[k]:
                h.wait()

    return body


def _tc_body(x_ref, p_ref, o_ref):
    o_ref[...] = x_ref[...] + p_ref[...]


def _tc_add(B, L, L1, D):
    bs = _TC_BS
    grid = (L1 // bs, B)
    return pl.pallas_call(
        _tc_body,
        grid=grid,
        in_specs=[
            pl.BlockSpec((1, bs, D), lambda i, b: (b, i, 0)),
            pl.BlockSpec((bs, D), lambda i, b: (i, 0)),
        ],
        out_specs=pl.BlockSpec((1, bs, D), lambda i, b: (b, i, 0)),
        out_shape=jax.ShapeDtypeStruct((B, L, D), jnp.float32),
    )


def kernel(x, pos_table):
    B, L, D = x.shape
    gran = _NW * _CH
    l2 = int(round(L * _SC_FRAC / gran)) * gran
    l1 = L - l2
    if l2 <= 0 or l1 <= 0 or l1 % _TC_BS != 0:
        l1, l2 = 0, L
    xf = x.reshape(B * L, D)
    sc_out = _sc_add(B, L, l1, l2, D)(xf, pos_table)
    sc_out = sc_out.reshape(B, l2, D)
    if l1 == 0:
        return sc_out
    tc_out = _tc_add(B, L, l1, D)(x, pos_table)
    return lax.dynamic_update_slice(tc_out, sc_out, (0, l1, 0))


# speedup vs baseline: 1.1063x; 1.1063x over previous
import functools

import jax
import jax.numpy as jnp
from jax import lax
from jax.experimental import pallas as pl
from jax.experimental.pallas import tpu as pltpu
from jax.experimental.pallas import tpu_sc as plsc

_NC, _NS, _LANES = 2, 16, 16
_NW = _NC * _NS

_NG = 3
_CH = 8
_UNROLL = 4


def _sc_add(nbatch, nseq, d):
    seq_per_w = nseq // _NW
    nchunk = seq_per_w // _CH
    assert d & (d - 1) == 0
    dshift = d.bit_length() - 1

    mesh = plsc.VectorSubcoreMesh(core_axis_name="c", subcore_axis_name="s")

    @functools.partial(
        pl.kernel,
        mesh=mesh,
        out_type=jax.ShapeDtypeStruct((nbatch * nseq, d), jnp.float32),
        scratch_types=[
            pltpu.VMEM((_NG, nbatch, _CH, d), jnp.float32),
            pltpu.VMEM((_NG, _CH, d), jnp.float32),
        ] + [pltpu.SemaphoreType.DMA] * (3 * _NG),
    )
    def body(x_hbm, p_hbm, o_hbm, xbuf, pbuf, *sems):
        lsems = sems[:_NG]
        psems = sems[_NG:2 * _NG]
        ssems = sems[2 * _NG:]
        c = lax.axis_index("c")
        s = lax.axis_index("s")
        wid = s * _NC + c
        base = wid * seq_per_w

        def issue_group(k):
            g = k % _NG
            hs = [pltpu.async_copy(p_hbm.at[pl.ds(base + k * _CH, _CH)],
                                   pbuf.at[g], psems[g])]
            for b in range(nbatch):
                roff = b * nseq + base + k * _CH
                hs.append(pltpu.async_copy(x_hbm.at[pl.ds(roff, _CH)],
                                           xbuf.at[g, b], lsems[g]))
            return hs

        loads = {k: issue_group(k) for k in range(min(_NG - 1, nchunk))}
        stores = {}
        for k in range(nchunk):
            g = k % _NG
            for h in loads.pop(k):
                h.wait()

            @plsc.parallel_loop(0, _CH * d, step=_LANES, unroll=_UNROLL)
            def cbody(o, g=g):
                r = o >> dshift
                sl = pl.ds(pl.multiple_of(o & (d - 1), _LANES), _LANES)
                p = pbuf[g, r, sl]
                for b in range(nbatch):
                    xbuf[g, b, r, sl] = xbuf[g, b, r, sl] + p

            shs = []
            for b in range(nbatch):
                roff = b * nseq + base + k * _CH
                shs.append(pltpu.async_copy(xbuf.at[g, b],
                                            o_hbm.at[pl.ds(roff, _CH)],
                                            ssems[g]))
            stores[k] = shs

            kn = k + _NG - 1
            if kn < nchunk:
                if k - 1 >= 0:
                    for h in stores.pop(k - 1):
                        h.wait()
                loads[kn] = issue_group(kn)
        for k in sorted(stores):
            for h in stores[k]:
                h.wait()

    return body


def kernel(x, pos_table):
    B, L, D = x.shape
    xf = x.reshape(B * L, D)
    out = _sc_add(B, L, D)(xf, pos_table)
    return out.reshape(B, L, D)
